# baseline (speedup 1.0000x reference)
import functools

import jax
import jax.numpy as jnp
from jax import lax
from jax.experimental import pallas as pl
from jax.experimental.pallas import tpu as pltpu
from jax.experimental.pallas import tpu_sc as plsc

B, T = 16384, 50
DIM = 32
NUM_TOKENS = B * T
NUM_ROWS = 1000000
NC, NS = 2, 16
NW = NC * NS
PER_W = NUM_TOKENS // NW
CHUNK = 128
CHUNKS = PER_W // CHUNK
K = 10
ROWS_G = K * CHUNK
G = CHUNKS // K
G2 = G // 2

W_BLK = 800
N_BLK = NUM_ROWS // W_BLK

_mesh = plsc.VectorSubcoreMesh(core_axis_name="c", subcore_axis_name="s")
_params = pltpu.CompilerParams(use_tc_tiling_on_sc=False)
_params_nolayout = pltpu.CompilerParams(
    use_tc_tiling_on_sc=False, needs_layout_passes=False)


@functools.partial(
    pl.kernel,
    mesh=_mesh,
    out_type=jax.ShapeDtypeStruct((NUM_ROWS, DIM), jnp.float32),
    scratch_types=[
        pltpu.VMEM((DIM, W_BLK), jnp.float32),
        pltpu.VMEM((W_BLK, DIM), jnp.float32),
    ],
    compiler_params=_params_nolayout,
)
def _transpose(wt_hbm, tbl_hbm, in_v, out_v):
    wid = lax.axis_index("s") * NC + lax.axis_index("c")
    lanes = lax.iota(jnp.int32, 16)

    def do_block(blk, carry):
        c0 = blk * W_BLK
        pltpu.sync_copy(wt_hbm.at[:, pl.ds(c0, W_BLK)], in_v)

        def do_rows(r0, carry2):
            rows = lanes + r0 * 16
            for f in range(DIM):
                v = in_v[f, pl.ds(r0 * 16, 16)]
                cols = jnp.full((16,), f, jnp.int32)
                plsc.store_scatter(out_v, [rows, cols], v)
            return carry2

        lax.fori_loop(0, W_BLK // 16, do_rows, 0)
        pltpu.sync_copy(out_v, tbl_hbm.at[pl.ds(c0, W_BLK)])
        return carry

    n_mine = (N_BLK - wid + NW - 1) // NW

    def body(k, carry):
        return do_block(wid + k * NW, carry)

    lax.fori_loop(0, n_mine, body, 0)


@functools.partial(
    pl.kernel,
    mesh=_mesh,
    out_type=jax.ShapeDtypeStruct((NUM_TOKENS, DIM), jnp.float32),
    scratch_types=[
        pltpu.VMEM((CHUNKS, CHUNK), jnp.int32),
        pltpu.VMEM((ROWS_G, DIM), jnp.float32),
        pltpu.VMEM((ROWS_G, DIM), jnp.float32),
        pltpu.SemaphoreType.DMA,
        pltpu.SemaphoreType.DMA,
    ],
    compiler_params=_params,
)
def _gather(tok_hbm, table_hbm, out_hbm, idx_v, rows0, rows1, gsem, ssem):
    wid = lax.axis_index("s") * NC + lax.axis_index("c")
    pltpu.sync_copy(tok_hbm.at[pl.ds(wid * CHUNKS, CHUNKS)], idx_v)
    base = wid * PER_W

    def fire_gathers(g, rows_v):
        handles = []
        for j in range(K):
            handles.append(pltpu.async_copy(
                table_hbm.at[idx_v.at[g * K + j]],
                rows_v.at[pl.ds(j * CHUNK, CHUNK)],
                gsem))
        for h in handles:
            h.wait()

    def body(p, carry):
        a = 2 * p
        b = a + 1
        fire_gathers(a, rows0)
        sc_a = pltpu.async_copy(
            rows0, out_hbm.at[pl.ds(base + a * ROWS_G, ROWS_G)], ssem)
        fire_gathers(b, rows1)
        sc_b = pltpu.async_copy(
            rows1, out_hbm.at[pl.ds(base + b * ROWS_G, ROWS_G)], ssem)
        sc_a.wait()
        sc_b.wait()
        return carry

    lax.fori_loop(0, G2, body, 0)


def kernel(token_ids, weight):
    tbl = _transpose(weight.T)
    tok = token_ids.reshape(NUM_TOKENS // CHUNK, CHUNK).astype(jnp.int32)
    out = _gather(tok, tbl)
    return out.reshape(B, T, DIM)


# speedup vs baseline: 1.9391x; 1.9391x over previous
import functools

import jax
import jax.numpy as jnp
from jax import lax
from jax.experimental import pallas as pl
from jax.experimental.pallas import tpu as pltpu
from jax.experimental.pallas import tpu_sc as plsc

B, T = 16384, 50
DIM = 32
NUM_TOKENS = B * T
NUM_ROWS = 1000000
PACK = 128 // DIM
NUM_PROWS = NUM_ROWS // PACK
NC, NS = 2, 16
NW = NC * NS
PER_W = NUM_TOKENS // NW
CHUNK = 128
CHUNKS = PER_W // CHUNK
PAIR = 2 * CHUNK
PAIRS = CHUNKS // 2

W_BLK = 1024
N_BLK = NUM_ROWS // W_BLK
W_REM = NUM_ROWS - N_BLK * W_BLK
P_BLK = W_BLK // PACK

_mesh = plsc.VectorSubcoreMesh(core_axis_name="c", subcore_axis_name="s")
_params = pltpu.CompilerParams(
    use_tc_tiling_on_sc=True, needs_layout_passes=False)


@functools.partial(
    pl.kernel,
    mesh=_mesh,
    out_type=jax.ShapeDtypeStruct((NUM_PROWS, 128), jnp.float32),
    scratch_types=[
        pltpu.VMEM((DIM, W_BLK), jnp.float32),
        pltpu.VMEM((P_BLK, 128), jnp.float32),
    ],
    compiler_params=_params,
)
def _transpose(wt_hbm, rem_hbm, tbl_hbm, in_v, out_v):
    wid = lax.axis_index("s") * NC + lax.axis_index("c")
    lanes = lax.iota(jnp.int32, 16)

    def transpose_cols(width):
        def do_rows(r0, carry2):
            flat0 = (lanes + r0 * 16) * DIM
            for f in range(DIM):
                v = in_v[f, pl.ds(r0 * 16, 16)]
                flat = flat0 + f
                plsc.store_scatter(
                    out_v,
                    [lax.shift_right_logical(flat, 7),
                     lax.bitwise_and(flat, 127)],
                    v)
            return carry2

        lax.fori_loop(0, width // 16, do_rows, 0)

    def do_block(blk, carry):
        c0 = blk * W_BLK
        pltpu.sync_copy(wt_hbm.at[:, pl.ds(c0, W_BLK)], in_v)
        transpose_cols(W_BLK)
        pltpu.sync_copy(out_v, tbl_hbm.at[pl.ds(blk * P_BLK, P_BLK)])
        return carry

    n_mine = (N_BLK - wid + NW - 1) // NW

    def body(k, carry):
        return do_block(wid + k * NW, carry)

    lax.fori_loop(0, n_mine, body, 0)

    @pl.when(wid == NW - 1)
    def _():
        pltpu.sync_copy(rem_hbm, out_v.at[pl.ds(0, W_REM // PACK)])
        pltpu.sync_copy(out_v.at[pl.ds(0, W_REM // PACK)],
                        tbl_hbm.at[pl.ds(N_BLK * P_BLK, W_REM // PACK)])


@functools.partial(
    pl.kernel,
    mesh=_mesh,
    out_type=jax.ShapeDtypeStruct((NUM_TOKENS // PACK, 128), jnp.float32),
    scratch_types=[
        pltpu.VMEM((CHUNKS, CHUNK), jnp.int32),
        pltpu.VMEM((2, CHUNK), jnp.int32),
        pltpu.VMEM((2, CHUNK), jnp.int32),
        pltpu.VMEM((PAIR, 128), jnp.float32),
        pltpu.VMEM((PAIR, 128), jnp.float32),
        pltpu.VMEM((PAIR // PACK, 128), jnp.float32),
        pltpu.VMEM((PAIR // PACK, 128), jnp.float32),
        pltpu.SemaphoreType.DMA,
        pltpu.SemaphoreType.DMA,
    ],
    compiler_params=_params,
)
def _gather(tok_hbm, table_hbm, out_hbm, idx_v, pidx_a, pidx_b,
            rows_a, rows_b, outc_a, outc_b, gsem, ssem):
    wid = lax.axis_index("s") * NC + lax.axis_index("c")
    pltpu.sync_copy(tok_hbm.at[pl.ds(wid * CHUNKS, CHUNKS)], idx_v)
    pbase = wid * (PER_W // PACK)
    lanes = lax.iota(jnp.int32, 16)

    def fire_pair(pair, pidx_v, rows_v):
        c = 2 * pair
        for half in range(2):
            for r0 in range(CHUNK // 16):
                t16 = idx_v[c + half, pl.ds(r0 * 16, 16)]
                pidx_v[half, pl.ds(r0 * 16, 16)] = (
                    lax.shift_right_logical(t16, 2))
        pltpu.async_copy(
            table_hbm.at[pidx_v.at[0]], rows_v.at[pl.ds(0, CHUNK)], gsem)
        pltpu.async_copy(
            table_hbm.at[pidx_v.at[1]], rows_v.at[pl.ds(CHUNK, CHUNK)],
            gsem)

    def extract(pair, rows_v, outc_v):
        c = 2 * pair
        for half in range(2):
            for r0 in range(CHUNK // 16):
                rows16 = lanes + (half * CHUNK + r0 * 16)
                t16 = idx_v[c + half, pl.ds(r0 * 16, 16)]
                sub16 = lax.bitwise_and(t16, 3) * DIM
                flat0 = rows16 * DIM
                for f in range(DIM):
                    v = plsc.load_gather(rows_v, [rows16, sub16 + f])
                    flat = flat0 + f
                    plsc.store_scatter(
                        outc_v,
                        [lax.shift_right_logical(flat, 7),
                         lax.bitwise_and(flat, 127)],
                        v)

    def scatter(pair, outc_v):
        return pltpu.async_copy(
            outc_v,
            out_hbm.at[pl.ds(pbase + pair * (PAIR // PACK), PAIR // PACK)],
            ssem)

    def drain2(rows_v):
        pltpu.make_async_copy(
            table_hbm.at[pl.ds(0, CHUNK)], rows_v.at[pl.ds(0, CHUNK)],
            gsem).wait()
        pltpu.make_async_copy(
            table_hbm.at[pl.ds(0, CHUNK)], rows_v.at[pl.ds(CHUNK, CHUNK)],
            gsem).wait()

    fire_pair(0, pidx_a, rows_a)
    fire_pair(1, pidx_b, rows_b)

    def body(q, carry):
        pa = 2 * q
        pb = pa + 1
        drain2(rows_a)
        extract(pa, rows_a, outc_a)
        sc_a = scatter(pa, outc_a)
        drain2(rows_b)
        extract(pb, rows_b, outc_b)
        sc_b = scatter(pb, outc_b)
        sc_a.wait()
        fire_pair(pa + 2, pidx_a, rows_a)
        sc_b.wait()
        fire_pair(pb + 2, pidx_b, rows_b)
        return carry

    lax.fori_loop(0, PAIRS // 2 - 1, body, 0)

    drain2(rows_a)
    extract(PAIRS - 2, rows_a, outc_a)
    sc_a = scatter(PAIRS - 2, outc_a)
    drain2(rows_b)
    extract(PAIRS - 1, rows_b, outc_b)
    sc_b = scatter(PAIRS - 1, outc_b)
    sc_a.wait()
    sc_b.wait()


def kernel(token_ids, weight):
    rem = weight[N_BLK * W_BLK:, :].reshape(W_REM // PACK, 128)
    tbl = _transpose(weight.T, rem)
    tok = token_ids.reshape(NUM_TOKENS // CHUNK, CHUNK).astype(jnp.int32)
    out = _gather(tok, tbl)
    return out.reshape(B, T, DIM)


# speedup vs baseline: 2.1303x; 1.0986x over previous
import functools

import jax
import jax.numpy as jnp
from jax import lax
from jax.experimental import pallas as pl
from jax.experimental.pallas import tpu as pltpu
from jax.experimental.pallas import tpu_sc as plsc

B, T = 16384, 50
DIM = 32
NUM_TOKENS = B * T
NUM_ROWS = 1000000
PACK = 128 // DIM
NUM_PROWS = NUM_ROWS // PACK
NC, NS = 2, 16
NW = NC * NS
PER_W = NUM_TOKENS // NW
CHUNK = 128
CHUNKS = PER_W // CHUNK
PAIR = 2 * CHUNK
PAIRS = CHUNKS // 2

W_BLK = 1024
N_BLK = NUM_ROWS // W_BLK
W_REM = NUM_ROWS - N_BLK * W_BLK
P_BLK = W_BLK // PACK
T_UNIF = (N_BLK + NW - 1) // NW

_mesh = plsc.VectorSubcoreMesh(core_axis_name="c", subcore_axis_name="s")
_params = pltpu.CompilerParams(
    use_tc_tiling_on_sc=True, needs_layout_passes=False)


@functools.partial(
    pl.kernel,
    mesh=_mesh,
    out_type=jax.ShapeDtypeStruct((NUM_PROWS, 128), jnp.float32),
    scratch_types=[
        pltpu.VMEM((DIM, W_BLK), jnp.float32),
        pltpu.VMEM((DIM, W_BLK), jnp.float32),
        pltpu.VMEM((P_BLK, 128), jnp.float32),
        pltpu.SemaphoreType.DMA,
    ],
    compiler_params=_params,
)
def _transpose(wt_hbm, rem_hbm, tbl_hbm, in_a, in_b, out_v, tsem):
    wid = lax.axis_index("s") * NC + lax.axis_index("c")
    lanes = lax.iota(jnp.int32, 16)

    def blk_of(k):
        return jnp.minimum(wid + k * NW, N_BLK - 1)

    def fire_in(k, in_v):
        return pltpu.async_copy(
            wt_hbm.at[:, pl.ds(blk_of(k) * W_BLK, W_BLK)], in_v, tsem)

    def drain_in(in_v):
        pltpu.make_async_copy(
            wt_hbm.at[:, pl.ds(0, W_BLK)], in_v, tsem).wait()

    def transpose_block(k, in_v):
        def do_rows(r0, carry2):
            flat0 = (lanes + r0 * 16) * DIM
            for f in range(DIM):
                v = in_v[f, pl.ds(r0 * 16, 16)]
                flat = flat0 + f
                plsc.store_scatter(
                    out_v,
                    [lax.shift_right_logical(flat, 7),
                     lax.bitwise_and(flat, 127)],
                    v)
            return carry2

        lax.fori_loop(0, W_BLK // 16, do_rows, 0)
        pltpu.sync_copy(out_v, tbl_hbm.at[pl.ds(blk_of(k) * P_BLK, P_BLK)])

    fire_in(0, in_a)

    def body(m, carry):
        ka = 2 * m
        drain_in(in_a)
        fire_in(ka + 1, in_b)
        transpose_block(ka, in_a)
        drain_in(in_b)
        fire_in(ka + 2, in_a)
        transpose_block(ka + 1, in_b)
        return carry

    lax.fori_loop(0, T_UNIF // 2, body, 0)
    drain_in(in_a)
    transpose_block(T_UNIF - 1, in_a)

    @pl.when(wid == NW - 1)
    def _():
        pltpu.sync_copy(rem_hbm, out_v.at[pl.ds(0, W_REM // PACK)])
        pltpu.sync_copy(out_v.at[pl.ds(0, W_REM // PACK)],
                        tbl_hbm.at[pl.ds(N_BLK * P_BLK, W_REM // PACK)])


@functools.partial(
    pl.kernel,
    mesh=_mesh,
    out_type=jax.ShapeDtypeStruct((NUM_TOKENS // PACK, 128), jnp.float32),
    scratch_types=[
        pltpu.VMEM((CHUNKS, CHUNK), jnp.int32),
        pltpu.VMEM((2, CHUNK), jnp.int32),
        pltpu.VMEM((2, CHUNK), jnp.int32),
        pltpu.VMEM((PAIR, 128), jnp.float32),
        pltpu.VMEM((PAIR, 128), jnp.float32),
        pltpu.VMEM((PAIR // PACK, 128), jnp.float32),
        pltpu.VMEM((PAIR // PACK, 128), jnp.float32),
        pltpu.SemaphoreType.DMA,
        pltpu.SemaphoreType.DMA,
    ],
    compiler_params=_params,
)
def _gather(tok_hbm, table_hbm, out_hbm, idx_v, pidx_a, pidx_b,
            rows_a, rows_b, outc_a, outc_b, gsem, ssem):
    wid = lax.axis_index("s") * NC + lax.axis_index("c")
    pltpu.sync_copy(tok_hbm.at[pl.ds(wid * CHUNKS, CHUNKS)], idx_v)
    pbase = wid * (PER_W // PACK)
    lanes = lax.iota(jnp.int32, 16)

    def fire_pair(pair, pidx_v, rows_v):
        c = 2 * pair

        def prep(g, carry):
            half = lax.shift_right_logical(g, 3)
            col0 = lax.bitwise_and(g, 7) * 16
            t16 = idx_v[c + half, pl.ds(col0, 16)]
            pidx_v[half, pl.ds(col0, 16)] = lax.shift_right_logical(t16, 2)
            return carry

        lax.fori_loop(0, PAIR // 16, prep, 0)
        pltpu.async_copy(
            table_hbm.at[pidx_v.at[0]], rows_v.at[pl.ds(0, CHUNK)], gsem)
        pltpu.async_copy(
            table_hbm.at[pidx_v.at[1]], rows_v.at[pl.ds(CHUNK, CHUNK)],
            gsem)

    def extract(pair, rows_v, outc_v):
        c = 2 * pair

        def group(g, carry):
            half = lax.shift_right_logical(g, 3)
            col0 = lax.bitwise_and(g, 7) * 16
            rows16 = lanes + g * 16
            t16 = idx_v[c + half, pl.ds(col0, 16)]
            sub16 = lax.bitwise_and(t16, 3) * DIM
            flat0 = rows16 * DIM
            for f in range(DIM):
                v = plsc.load_gather(rows_v, [rows16, sub16 + f])
                flat = flat0 + f
                plsc.store_scatter(
                    outc_v,
                    [lax.shift_right_logical(flat, 7),
                     lax.bitwise_and(flat, 127)],
                    v)
            return carry

        lax.fori_loop(0, PAIR // 16, group, 0)

    def scatter(pair, outc_v):
        pltpu.async_copy(
            outc_v,
            out_hbm.at[pl.ds(pbase + pair * (PAIR // PACK), PAIR // PACK)],
            ssem)

    def drain2(rows_v):
        pltpu.make_async_copy(
            table_hbm.at[pl.ds(0, CHUNK)], rows_v.at[pl.ds(0, CHUNK)],
            gsem).wait()
        pltpu.make_async_copy(
            table_hbm.at[pl.ds(0, CHUNK)], rows_v.at[pl.ds(CHUNK, CHUNK)],
            gsem).wait()

    def drain_scat(outc_v):
        pltpu.make_async_copy(
            out_hbm.at[pl.ds(0, PAIR // PACK)], outc_v, ssem).wait()

    fire_pair(0, pidx_a, rows_a)
    fire_pair(1, pidx_b, rows_b)
    drain2(rows_a)
    extract(0, rows_a, outc_a)
    fire_pair(2, pidx_a, rows_a)
    scatter(0, outc_a)
    drain2(rows_b)
    extract(1, rows_b, outc_b)
    fire_pair(3, pidx_b, rows_b)
    scatter(1, outc_b)

    def body(q, carry):
        pa = 2 * q
        pb = pa + 1
        drain_scat(outc_a)
        drain2(rows_a)
        extract(pa, rows_a, outc_a)
        fire_pair(pa + 2, pidx_a, rows_a)
        scatter(pa, outc_a)
        drain_scat(outc_b)
        drain2(rows_b)
        extract(pb, rows_b, outc_b)
        fire_pair(pb + 2, pidx_b, rows_b)
        scatter(pb, outc_b)
        return carry

    lax.fori_loop(1, PAIRS // 2 - 1, body, 0)

    drain_scat(outc_a)
    drain2(rows_a)
    extract(PAIRS - 2, rows_a, outc_a)
    scatter(PAIRS - 2, outc_a)
    drain_scat(outc_b)
    drain2(rows_b)
    extract(PAIRS - 1, rows_b, outc_b)
    scatter(PAIRS - 1, outc_b)
    drain_scat(outc_a)
    drain_scat(outc_b)


def kernel(token_ids, weight):
    rem = weight[N_BLK * W_BLK:, :].reshape(W_REM // PACK, 128)
    tbl = _transpose(weight.T, rem)
    tok = token_ids.reshape(NUM_TOKENS // CHUNK, CHUNK).astype(jnp.int32)
    out = _gather(tok, tbl)
    return out.reshape(B, T, DIM)


# speedup vs baseline: 2.4599x; 1.1547x over previous
import functools

import jax
import jax.numpy as jnp
from jax import lax
from jax.experimental import pallas as pl
from jax.experimental.pallas import tpu as pltpu
from jax.experimental.pallas import tpu_sc as plsc

B, T = 16384, 50
DIM = 32
NUM_TOKENS = B * T
NUM_ROWS = 1000000
NC, NS = 2, 16
NW = NC * NS
PER_W = NUM_TOKENS // NW
CHUNK = 128
CHUNKS = PER_W // CHUNK
K = 10
ROWS_G = K * CHUNK
G = CHUNKS // K
G2 = G // 2

W_BLK = 512
N_BLK = NUM_ROWS // W_BLK
W_REM = NUM_ROWS - N_BLK * W_BLK
REM_ELEMS = W_REM * DIM
BLK_ELEMS = W_BLK * DIM
T_UNIF = (N_BLK + NW - 1) // NW

_mesh = plsc.VectorSubcoreMesh(core_axis_name="c", subcore_axis_name="s")
_params_tiled = pltpu.CompilerParams(
    use_tc_tiling_on_sc=True, needs_layout_passes=False)
_params_linear = pltpu.CompilerParams(use_tc_tiling_on_sc=False)


@functools.partial(
    pl.kernel,
    mesh=_mesh,
    out_type=jax.ShapeDtypeStruct((NUM_ROWS * DIM,), jnp.float32),
    scratch_types=[
        pltpu.VMEM((DIM, W_BLK), jnp.float32),
        pltpu.VMEM((DIM, W_BLK), jnp.float32),
        pltpu.VMEM((BLK_ELEMS,), jnp.float32),
        pltpu.VMEM((BLK_ELEMS,), jnp.float32),
        pltpu.SemaphoreType.DMA,
        pltpu.SemaphoreType.DMA,
    ],
    compiler_params=_params_tiled,
)
def _transpose(wt_hbm, rem_hbm, tbl_hbm, in_a, in_b, out_a, out_b,
               tsem, osem):
    wid = lax.axis_index("s") * NC + lax.axis_index("c")
    lanes = lax.iota(jnp.int32, 16)

    def blk_of(k):
        return jnp.minimum(wid + k * NW, N_BLK - 1)

    def fire_in(k, in_v):
        pltpu.async_copy(
            wt_hbm.at[:, pl.ds(blk_of(k) * W_BLK, W_BLK)], in_v, tsem)

    def drain_in(in_v):
        pltpu.make_async_copy(
            wt_hbm.at[:, pl.ds(0, W_BLK)], in_v, tsem).wait()

    def drain_out(out_v):
        pltpu.make_async_copy(
            tbl_hbm.at[pl.ds(0, BLK_ELEMS)], out_v, osem).wait()

    def transpose_block(in_v, out_v):
        def do_rows(r0, carry2):
            flat0 = (lanes + r0 * 16) * DIM
            for f in range(DIM):
                v = in_v[f, pl.ds(r0 * 16, 16)]
                plsc.store_scatter(out_v, [flat0 + f], v)
            return carry2

        lax.fori_loop(0, W_BLK // 16, do_rows, 0)

    def scat_out(k, out_v):
        pltpu.async_copy(
            out_v, tbl_hbm.at[pl.ds(blk_of(k) * BLK_ELEMS, BLK_ELEMS)],
            osem)

    fire_in(0, in_a)
    fire_in(1, in_b)
    drain_in(in_a)
    transpose_block(in_a, out_a)
    fire_in(2, in_a)
    scat_out(0, out_a)
    drain_in(in_b)
    transpose_block(in_b, out_b)
    fire_in(3, in_b)
    scat_out(1, out_b)

    def body(m, carry):
        ka = 2 * m
        kb = ka + 1
        drain_out(out_a)
        drain_in(in_a)
        transpose_block(in_a, out_a)
        fire_in(ka + 2, in_a)
        scat_out(ka, out_a)
        drain_out(out_b)
        drain_in(in_b)
        transpose_block(in_b, out_b)
        fire_in(kb + 2, in_b)
        scat_out(kb, out_b)
        return carry

    lax.fori_loop(1, T_UNIF // 2 - 1, body, 0)

    drain_out(out_a)
    drain_in(in_a)
    transpose_block(in_a, out_a)
    scat_out(T_UNIF - 2, out_a)
    drain_out(out_b)
    drain_in(in_b)
    transpose_block(in_b, out_b)
    scat_out(T_UNIF - 1, out_b)
    drain_out(out_a)
    drain_out(out_b)

    @pl.when(wid == NW - 1)
    def _():
        pltpu.sync_copy(rem_hbm, out_a.at[pl.ds(0, REM_ELEMS)])
        pltpu.sync_copy(out_a.at[pl.ds(0, REM_ELEMS)],
                        tbl_hbm.at[pl.ds(N_BLK * BLK_ELEMS, REM_ELEMS)])


@functools.partial(
    pl.kernel,
    mesh=_mesh,
    out_type=jax.ShapeDtypeStruct((NUM_TOKENS, DIM), jnp.float32),
    scratch_types=[
        pltpu.VMEM((CHUNKS, CHUNK), jnp.int32),
        pltpu.VMEM((ROWS_G, DIM), jnp.float32),
        pltpu.VMEM((ROWS_G, DIM), jnp.float32),
        pltpu.SemaphoreType.DMA,
        pltpu.SemaphoreType.DMA,
    ],
    compiler_params=_params_linear,
)
def _gather(tok_hbm, table_hbm, out_hbm, idx_v, rows0, rows1, gsem, ssem):
    wid = lax.axis_index("s") * NC + lax.axis_index("c")
    pltpu.sync_copy(tok_hbm.at[pl.ds(wid * CHUNKS, CHUNKS)], idx_v)
    base = wid * PER_W

    def fire_gathers(g, rows_v):
        handles = []
        for j in range(K):
            handles.append(pltpu.async_copy(
                table_hbm.at[idx_v.at[g * K + j]],
                rows_v.at[pl.ds(j * CHUNK, CHUNK)],
                gsem))
        for h in handles:
            h.wait()

    def body(p, carry):
        a = 2 * p
        b = a + 1
        fire_gathers(a, rows0)
        sc_a = pltpu.async_copy(
            rows0, out_hbm.at[pl.ds(base + a * ROWS_G, ROWS_G)], ssem)
        fire_gathers(b, rows1)
        sc_b = pltpu.async_copy(
            rows1, out_hbm.at[pl.ds(base + b * ROWS_G, ROWS_G)], ssem)
        sc_a.wait()
        sc_b.wait()
        return carry

    lax.fori_loop(0, G2, body, 0)


def kernel(token_ids, weight):
    rem = weight[N_BLK * W_BLK:, :].reshape(REM_ELEMS)
    tbl = _transpose(weight.T, rem).reshape(NUM_ROWS, DIM)
    tok = token_ids.reshape(NUM_TOKENS // CHUNK, CHUNK).astype(jnp.int32)
    out = _gather(tok, tbl)
    return out.reshape(B, T, DIM)


# speedup vs baseline: 2.6523x; 1.0782x over previous
import functools

import jax
import jax.numpy as jnp
from jax import lax
from jax.experimental import pallas as pl
from jax.experimental.pallas import tpu as pltpu
from jax.experimental.pallas import tpu_sc as plsc

B, T = 16384, 50
DIM = 32
NUM_TOKENS = B * T
NC, NS = 2, 16
NW = NC * NS
PER_W = NUM_TOKENS // NW
CHUNK = 128
CHUNKS = PER_W // CHUNK
K = 10
ROWS_G = K * CHUNK
G = CHUNKS // K
G2 = G // 2

_mesh = plsc.VectorSubcoreMesh(core_axis_name="c", subcore_axis_name="s")


@functools.partial(
    pl.kernel,
    mesh=_mesh,
    out_type=jax.ShapeDtypeStruct((NUM_TOKENS, DIM), jnp.float32),
    scratch_types=[
        pltpu.VMEM((CHUNKS, CHUNK), jnp.int32),
        pltpu.VMEM((ROWS_G, DIM), jnp.float32),
        pltpu.VMEM((ROWS_G, DIM), jnp.float32),
        pltpu.SemaphoreType.DMA,
        pltpu.SemaphoreType.DMA,
    ],
    compiler_params=pltpu.CompilerParams(use_tc_tiling_on_sc=False),
)
def _embed(tok_hbm, table_hbm, out_hbm, idx_v, rows0, rows1, gsem, ssem):
    wid = lax.axis_index("s") * NC + lax.axis_index("c")
    pltpu.sync_copy(tok_hbm.at[pl.ds(wid * CHUNKS, CHUNKS)], idx_v)
    base = wid * PER_W

    def fire_gathers(g, rows_v):
        handles = []
        for j in range(K):
            handles.append(pltpu.async_copy(
                table_hbm.at[idx_v.at[g * K + j]],
                rows_v.at[pl.ds(j * CHUNK, CHUNK)],
                gsem))
        for h in handles:
            h.wait()

    def body(p, carry):
        a = 2 * p
        b = a + 1
        fire_gathers(a, rows0)
        sc_a = pltpu.async_copy(
            rows0, out_hbm.at[pl.ds(base + a * ROWS_G, ROWS_G)], ssem)
        fire_gathers(b, rows1)
        sc_b = pltpu.async_copy(
            rows1, out_hbm.at[pl.ds(base + b * ROWS_G, ROWS_G)], ssem)
        sc_a.wait()
        sc_b.wait()
        return carry

    lax.fori_loop(0, G2, body, 0)


def kernel(token_ids, weight):
    tok = token_ids.reshape(NUM_TOKENS // CHUNK, CHUNK).astype(jnp.int32)
    out = _embed(tok, weight)
    return out.reshape(B, T, DIM)
